# Initial kernel scaffold; baseline (speedup 1.0000x reference)
import functools

import jax
import jax.numpy as jnp
from jax import lax
from jax.experimental import pallas as pl
from jax.experimental.pallas import tpu as pltpu
from jax.experimental.pallas import tpu_sc as plsc

_INFO = plsc.get_sparse_core_info()
_NC = _INFO.num_cores
_NS = _INFO.num_subcores
_NW = _NC * _NS

_CHUNK = 128


def _make_gather(n_rows: int, d: int):
    assert n_rows % (_NW * _CHUNK) == 0
    per_w = n_rows // _NW
    n_chunks = per_w // _CHUNK

    mesh = plsc.VectorSubcoreMesh(core_axis_name="c", subcore_axis_name="s")

    @functools.partial(
        pl.kernel,
        out_type=jax.ShapeDtypeStruct((n_rows, d), jnp.float32),
        mesh=mesh,
        scratch_types=[
            pltpu.VMEM((n_chunks, _CHUNK), jnp.int32),
            pltpu.VMEM((_CHUNK, d), jnp.float32),
            pltpu.SemaphoreType.DMA,
        ],
    )
    def gather_kernel(table_hbm, idx_hbm, out_hbm, idx_v, rows_v, sem):
        wid = lax.axis_index("s") * _NC + lax.axis_index("c")
        pltpu.sync_copy(idx_hbm.at[pl.ds(wid * n_chunks, n_chunks)], idx_v)
        base = wid * per_w

        def body(j, carry):
            pltpu.async_copy(table_hbm.at[idx_v.at[j]], rows_v, sem).wait()
            pltpu.sync_copy(rows_v, out_hbm.at[pl.ds(base + j * _CHUNK, _CHUNK)])
            return carry

        lax.fori_loop(0, n_chunks, body, 0)

    return gather_kernel


def kernel(input, table):
    b, s = input.shape
    v, d = table.shape
    n_rows = b * s
    idx2d = input.reshape(n_rows // _CHUNK, _CHUNK).astype(jnp.int32)
    out = _make_gather(n_rows, d)(table, idx2d)
    return out.reshape(b, s, d)


# speedup vs baseline: 4.0900x; 4.0900x over previous
import functools

import jax
import jax.numpy as jnp
from jax import lax
from jax.experimental import pallas as pl
from jax.experimental.pallas import tpu as pltpu
from jax.experimental.pallas import tpu_sc as plsc

_INFO = plsc.get_sparse_core_info()
_NC = _INFO.num_cores
_NS = _INFO.num_subcores
_NW = _NC * _NS

_CHUNK = 128


def _make_gather(n_rows: int, d: int):
    assert n_rows % (_NW * _CHUNK) == 0
    per_w = n_rows // _NW
    n_chunks = per_w // _CHUNK

    mesh = plsc.VectorSubcoreMesh(core_axis_name="c", subcore_axis_name="s")

    @functools.partial(
        pl.kernel,
        out_type=jax.ShapeDtypeStruct((n_rows, d), jnp.float32),
        mesh=mesh,
        scratch_types=[
            pltpu.VMEM((n_chunks, _CHUNK), jnp.int32),
            pltpu.VMEM((_CHUNK, d), jnp.float32),
            pltpu.SemaphoreType.DMA,
        ],
        compiler_params=pltpu.CompilerParams(use_tc_tiling_on_sc=False),
    )
    def gather_kernel(table_hbm, idx_hbm, out_hbm, idx_v, rows_v, sem):
        wid = lax.axis_index("s") * _NC + lax.axis_index("c")
        pltpu.sync_copy(idx_hbm.at[wid], idx_v)
        base = wid * per_w

        def body(j, carry):
            pltpu.async_copy(table_hbm.at[idx_v.at[j]], rows_v, sem).wait()
            pltpu.sync_copy(rows_v, out_hbm.at[pl.ds(base + j * _CHUNK, _CHUNK)])
            return carry

        lax.fori_loop(0, n_chunks, body, 0)

    return gather_kernel


def kernel(input, table):
    b, s = input.shape
    v, d = table.shape
    n_rows = b * s
    idx3d = input.reshape(_NW, n_rows // (_NW * _CHUNK), _CHUNK).astype(jnp.int32)
    out = _make_gather(n_rows, d)(table, idx3d)
    return out.reshape(b, s, d)


# speedup vs baseline: 4.6622x; 1.1399x over previous
import functools

import jax
import jax.numpy as jnp
from jax import lax
from jax.experimental import pallas as pl
from jax.experimental.pallas import tpu as pltpu
from jax.experimental.pallas import tpu_sc as plsc

_INFO = plsc.get_sparse_core_info()
_NC = _INFO.num_cores
_NS = _INFO.num_subcores
_NW = _NC * _NS

_CHUNK = 320
_NBUF = 4
_LEAD = 2


def _make_gather(n_rows: int, d: int):
    per_w = n_rows // _NW
    n_chunks = per_w // _CHUNK
    assert per_w % _CHUNK == 0 and n_chunks % _NBUF == 0 and n_chunks >= 2 * _NBUF

    mesh = plsc.VectorSubcoreMesh(core_axis_name="c", subcore_axis_name="s")

    @functools.partial(
        pl.kernel,
        out_type=jax.ShapeDtypeStruct((n_rows, d), jnp.float32),
        mesh=mesh,
        scratch_types=(
            [pltpu.VMEM((n_chunks, _CHUNK), jnp.int32)]
            + [pltpu.VMEM((_CHUNK, d), jnp.float32) for _ in range(_NBUF)]
            + [pltpu.SemaphoreType.DMA for _ in range(2 * _NBUF)]
        ),
        compiler_params=pltpu.CompilerParams(use_tc_tiling_on_sc=False),
    )
    def gather_kernel(table_hbm, idx_hbm, out_hbm, idx_v, *bufs_and_sems):
        rows = bufs_and_sems[:_NBUF]
        sg = bufs_and_sems[_NBUF:2 * _NBUF]
        so = bufs_and_sems[2 * _NBUF:3 * _NBUF]

        wid = lax.axis_index("s") * _NC + lax.axis_index("c")
        pltpu.sync_copy(idx_hbm.at[wid], idx_v)
        base = wid * per_w

        def fire_gather(k, slot):
            pltpu.async_copy(table_hbm.at[idx_v.at[k]], rows[slot], sg[slot])

        def wait_gather(k, slot):
            pltpu.make_async_copy(
                table_hbm.at[idx_v.at[k]], rows[slot], sg[slot]).wait()

        def out_slice(c):
            return out_hbm.at[pl.ds(base + c * _CHUNK, _CHUNK)]

        def fire_copy(c, slot):
            pltpu.async_copy(rows[slot], out_slice(c), so[slot])

        def wait_copy(c, slot):
            pltpu.make_async_copy(rows[slot], out_slice(c), so[slot]).wait()

        def body(k, b, wait_prev_copy, fire_next):
            wait_gather(k, b)
            fire_copy(k, b)
            if fire_next:
                bn = (b + _LEAD) % _NBUF
                if wait_prev_copy:
                    wait_copy(k + _LEAD - _NBUF, bn)
                fire_gather(k + _LEAD, bn)

        for c in range(_LEAD):
            fire_gather(c, c)
        for b in range(_NBUF):
            body(b, b, wait_prev_copy=(b >= _NBUF - _LEAD), fire_next=True)

        @pl.loop(_NBUF, n_chunks - _NBUF, step=_NBUF)
        def _(g):
            for b in range(_NBUF):
                body(g + b, b, wait_prev_copy=True, fire_next=True)

        for b in range(_NBUF):
            k = n_chunks - _NBUF + b
            body(k, b, wait_prev_copy=True, fire_next=(b < _NBUF - _LEAD))
        for b in range(_NBUF):
            wait_copy(n_chunks - _NBUF + b, b)

    return gather_kernel


def kernel(input, table):
    b, s = input.shape
    v, d = table.shape
    n_rows = b * s
    idx3d = input.reshape(_NW, n_rows // _NW // _CHUNK, _CHUNK).astype(jnp.int32)
    out = _make_gather(n_rows, d)(table, idx3d)
    return out.reshape(b, s, d)
